# baseline (speedup 1.0000x reference)
import jax
import jax.numpy as jnp
from jax.experimental import pallas as pl
from jax.experimental.pallas import tpu as pltpu


def _fused_kernel(x_hbm, w_ref, wg_ref, thr_ref, bias_ref,
                  o_ref, g_ref, coef_ref, xf_ref, xb_ref, sem):
    e = pl.program_id(0)
    i = pl.program_id(1)
    BQ = xf_ref.shape[0]
    BM = o_ref.shape[0]
    C = w_ref.shape[1]
    E = coef_ref.shape[1]

    @pl.when(e == 0)
    def _stage_and_gate():
        for k in range(2):
            q = i * 2 + k
            cp = pltpu.make_async_copy(
                x_hbm.at[pl.ds(q * BQ, BQ)], xf_ref, sem)
            cp.start()
            cp.wait()
            xb_ref[pl.ds(q * BQ, BQ), :] = xf_ref[...].astype(jnp.bfloat16)
        xbi = xb_ref[pl.ds(i * BM, BM), :]
        logits = jax.lax.dot_general(
            xbi, wg_ref[...], (((1,), (1,)), ((), ())),
            preferred_element_type=jnp.float32)
        thr = jnp.transpose(thr_ref[...])
        g = (logits > thr).astype(jnp.float32)
        g_ref[...] = g
        act = float(C) * jnp.sum(g, axis=1, keepdims=True)
        denom = jnp.where(act > 0.0, act, 1.0)
        coef_ref[...] = g * (float(C) / denom)

    w = w_ref[0].astype(jnp.bfloat16)
    acc = jax.lax.dot_general(
        xb_ref[pl.ds(i * BM, BM), :], w, (((1,), (1,)), ((), ())),
        preferred_element_type=jnp.float32)
    onehot = (jax.lax.broadcasted_iota(jnp.int32, (1, E), 1) == e)
    c = jnp.sum(jnp.where(onehot, coef_ref[...], 0.0),
                axis=1, keepdims=True)
    o_ref[...] = (acc + bias_ref[0]) * c


def kernel(x, Wg_w, Wg_b, comp_w, comp_b):
    B, D = x.shape
    E, C, _ = comp_w.shape
    NB = 2
    BM = B // NB
    BQ = B // 4

    with jax.ensure_compile_time_eval():
        U = jax.random.uniform(jax.random.key(42), (B, E), jnp.float32)
        logitU = (jnp.log(U) - jnp.log1p(-U)).T
    thrT = logitU - Wg_b[:, None]

    wg = Wg_w.astype(jnp.bfloat16)
    bias3 = comp_b[:, None, :]

    out, G, _ = pl.pallas_call(
        _fused_kernel,
        grid=(E, NB),
        in_specs=[
            pl.BlockSpec(memory_space=pltpu.HBM),
            pl.BlockSpec((1, C, D), lambda e, i: (e, 0, 0)),
            pl.BlockSpec((E, D), lambda e, i: (0, 0)),
            pl.BlockSpec((E, BM), lambda e, i: (0, i)),
            pl.BlockSpec((1, 1, C), lambda e, i: (e, 0, 0)),
        ],
        out_specs=[
            pl.BlockSpec((BM, C), lambda e, i: (i, e)),
            pl.BlockSpec((BM, E), lambda e, i: (i, 0)),
            pl.BlockSpec((BM, E), lambda e, i: (i, 0)),
        ],
        out_shape=[
            jax.ShapeDtypeStruct((B, E * C), jnp.float32),
            jax.ShapeDtypeStruct((B, E), jnp.float32),
            jax.ShapeDtypeStruct((B, E), jnp.float32),
        ],
        scratch_shapes=[
            pltpu.VMEM((BQ, D), jnp.float32),
            pltpu.VMEM((B, D), jnp.bfloat16),
            pltpu.SemaphoreType.DMA,
        ],
        compiler_params=pltpu.CompilerParams(
            vmem_limit_bytes=66000000),
    )(x, comp_w, wg, thrT, bias3)
    return out, G


# speedup vs baseline: 1.0820x; 1.0820x over previous
import jax
import jax.numpy as jnp
from jax.experimental import pallas as pl
from jax.experimental.pallas import tpu as pltpu


def _fused_kernel(x_hbm, w_ref, wg_ref, thr_ref, bias_ref,
                  o_ref, g_ref, coef_ref, xf_ref, xb_ref, sem):
    i = pl.program_id(0)
    j = pl.program_id(1)
    NB = pl.num_programs(0)
    BM = xf_ref.shape[0]
    C = w_ref.shape[1]

    @pl.when(j == 0)
    def _stage_and_gate():
        @pl.when(i == 0)
        def _first():
            pltpu.make_async_copy(
                x_hbm.at[pl.ds(0, BM)], xf_ref, sem).start()
        pltpu.make_async_copy(
            x_hbm.at[pl.ds(i * BM, BM)], xf_ref, sem).wait()
        xb = xf_ref[...].astype(jnp.bfloat16)
        xb_ref[...] = xb
        logits = jax.lax.dot_general(
            xb, wg_ref[...], (((1,), (1,)), ((), ())),
            preferred_element_type=jnp.float32)
        thr = jnp.transpose(thr_ref[...])
        g = (logits > thr).astype(jnp.float32)
        g_ref[...] = g
        act = float(C) * jnp.sum(g, axis=1, keepdims=True)
        denom = jnp.where(act > 0.0, act, 1.0)
        coef_ref[...] = g * (float(C) / denom)

    @pl.when((j == 1) & (i + 1 < NB))
    def _prefetch_next():
        pltpu.make_async_copy(
            x_hbm.at[pl.ds((i + 1) * BM, BM)], xf_ref, sem).start()

    w = w_ref[0].astype(jnp.bfloat16)
    acc = jax.lax.dot_general(
        xb_ref[...], w, (((1,), (1,)), ((), ())),
        preferred_element_type=jnp.float32)
    E = coef_ref.shape[1]
    onehot = (jax.lax.broadcasted_iota(jnp.int32, (1, E), 1) == j)
    c = jnp.sum(jnp.where(onehot, coef_ref[...], 0.0),
                axis=1, keepdims=True)
    o_ref[...] = (acc + bias_ref[0]) * c


def kernel(x, Wg_w, Wg_b, comp_w, comp_b):
    B, D = x.shape
    E, C, _ = comp_w.shape
    NB = 2
    BM = B // NB

    with jax.ensure_compile_time_eval():
        U = jax.random.uniform(jax.random.key(42), (B, E), jnp.float32)
        logitU = (jnp.log(U) - jnp.log1p(-U)).T
    thrT = logitU - Wg_b[:, None]

    wg = Wg_w.astype(jnp.bfloat16)
    bias3 = comp_b[:, None, :]

    out, G, _ = pl.pallas_call(
        _fused_kernel,
        grid=(NB, E),
        in_specs=[
            pl.BlockSpec(memory_space=pltpu.HBM),
            pl.BlockSpec((1, C, D), lambda i, j: (j, 0, 0)),
            pl.BlockSpec((E, D), lambda i, j: (0, 0)),
            pl.BlockSpec((E, BM), lambda i, j: (0, i)),
            pl.BlockSpec((1, 1, C), lambda i, j: (j, 0, 0)),
        ],
        out_specs=[
            pl.BlockSpec((BM, C), lambda i, j: (i, j)),
            pl.BlockSpec((BM, E), lambda i, j: (i, 0)),
            pl.BlockSpec((BM, E), lambda i, j: (i, 0)),
        ],
        out_shape=[
            jax.ShapeDtypeStruct((B, E * C), jnp.float32),
            jax.ShapeDtypeStruct((B, E), jnp.float32),
            jax.ShapeDtypeStruct((B, E), jnp.float32),
        ],
        scratch_shapes=[
            pltpu.VMEM((BM, D), jnp.float32),
            pltpu.VMEM((BM, D), jnp.bfloat16),
            pltpu.SemaphoreType.DMA,
        ],
    )(x, comp_w, wg, thrT, bias3)
    return out, G


# speedup vs baseline: 1.0825x; 1.0005x over previous
import jax
import jax.numpy as jnp
from jax.experimental import pallas as pl
from jax.experimental.pallas import tpu as pltpu


def _fused_kernel(x_hbm, w_ref, wg_ref, thr_ref, bias_ref,
                  o_ref, g_ref, coef_ref, xf_ref, xb_ref, sem):
    i = pl.program_id(0)
    j = pl.program_id(1)
    NB = pl.num_programs(0)
    BM = xf_ref.shape[0]
    C = w_ref.shape[1]

    BQ = BM // 4

    def _block_copies(blk):
        return [pltpu.make_async_copy(
                    x_hbm.at[pl.ds(blk * BM + k * BQ, BQ)],
                    xf_ref.at[pl.ds(k * BQ, BQ)], sem)
                for k in range(4)]

    @pl.when(j == 0)
    def _stage_and_gate():
        @pl.when(i == 0)
        def _first():
            for cp in _block_copies(0):
                cp.start()
        for cp in _block_copies(i):
            cp.wait()
        xb = xf_ref[...].astype(jnp.bfloat16)
        xb_ref[...] = xb
        logits = jax.lax.dot_general(
            xb, wg_ref[...], (((1,), (1,)), ((), ())),
            preferred_element_type=jnp.float32)
        thr = jnp.transpose(thr_ref[...])
        g = (logits > thr).astype(jnp.float32)
        g_ref[...] = g
        act = float(C) * jnp.sum(g, axis=1, keepdims=True)
        denom = jnp.where(act > 0.0, act, 1.0)
        coef_ref[...] = g * (float(C) / denom)

    @pl.when((j == 1) & (i + 1 < NB))
    def _prefetch_next():
        for cp in _block_copies(i + 1):
            cp.start()

    w = w_ref[0].astype(jnp.bfloat16)
    acc = jax.lax.dot_general(
        xb_ref[...], w, (((1,), (1,)), ((), ())),
        preferred_element_type=jnp.float32)
    E = coef_ref.shape[1]
    onehot = (jax.lax.broadcasted_iota(jnp.int32, (1, E), 1) == j)
    c = jnp.sum(jnp.where(onehot, coef_ref[...], 0.0),
                axis=1, keepdims=True)
    o_ref[...] = (acc + bias_ref[0]) * c


def kernel(x, Wg_w, Wg_b, comp_w, comp_b):
    B, D = x.shape
    E, C, _ = comp_w.shape
    NB = 2
    BM = B // NB

    with jax.ensure_compile_time_eval():
        U = jax.random.uniform(jax.random.key(42), (B, E), jnp.float32)
        logitU = (jnp.log(U) - jnp.log1p(-U)).T
    thrT = logitU - Wg_b[:, None]

    wg = Wg_w.astype(jnp.bfloat16)
    bias3 = comp_b[:, None, :]

    out, G, _ = pl.pallas_call(
        _fused_kernel,
        grid=(NB, E),
        in_specs=[
            pl.BlockSpec(memory_space=pltpu.HBM),
            pl.BlockSpec((1, C, D), lambda i, j: (j, 0, 0)),
            pl.BlockSpec((E, D), lambda i, j: (0, 0)),
            pl.BlockSpec((E, BM), lambda i, j: (0, i)),
            pl.BlockSpec((1, 1, C), lambda i, j: (j, 0, 0)),
        ],
        out_specs=[
            pl.BlockSpec((BM, C), lambda i, j: (i, j)),
            pl.BlockSpec((BM, E), lambda i, j: (i, 0)),
            pl.BlockSpec((BM, E), lambda i, j: (i, 0)),
        ],
        out_shape=[
            jax.ShapeDtypeStruct((B, E * C), jnp.float32),
            jax.ShapeDtypeStruct((B, E), jnp.float32),
            jax.ShapeDtypeStruct((B, E), jnp.float32),
        ],
        scratch_shapes=[
            pltpu.VMEM((BM, D), jnp.float32),
            pltpu.VMEM((BM, D), jnp.bfloat16),
            pltpu.SemaphoreType.DMA,
        ],
    )(x, comp_w, wg, thrT, bias3)
    return out, G
